# baseline (speedup 1.0000x reference)
import functools

import jax
import jax.numpy as jnp
from jax.experimental import pallas as pl
from jax.experimental.pallas import tpu as pltpu

B, N, F, H = 2, 4096, 128, 128
BM = 512
NI = N // BM


NS = 2
BS = BM // NS


def _fused_kernel(adj_top_ref, adj_bot_ref, x_ref, w1_ref, b1_ref, w2_ref,
                  b2_ref, h2_ref, gh_ref, s1_scr, s2_scr, cache_scr):
    t = pl.program_id(0)

    @pl.when(t >= NI)
    def _():
        pb = jnp.where(t < 2 * NI, 0, 1)
        pi = jnp.where(t < 2 * NI, t - NI, t - 2 * NI)
        a = cache_scr[pl.ds(pi * BM, BM), :]
        tacc = jnp.dot(a, s2_scr[pb], preferred_element_type=jnp.float32)
        h2 = jnp.maximum(tacc + b2_ref[...], 0.0)
        h2_ref[...] = h2[None]
        part = jnp.sum(h2, axis=0, keepdims=True)
        row = 2 * pb + 1

        @pl.when(pi == 0)
        def _():
            gh_ref[pl.ds(row, 1), :] = part

        @pl.when(pi != 0)
        def _():
            gh_ref[pl.ds(row, 1), :] += part

    @pl.when(t < 2 * NI)
    def _():
        fb = jnp.where(t < NI, 0, 1)
        fi = jnp.where(t < NI, t, t - NI)

        @pl.when(fi == 0)
        def _():
            s1 = jnp.dot(
                x_ref[0], w1_ref[...], preferred_element_type=jnp.float32
            )
            s1_scr[...] = s1.astype(jnp.bfloat16)

        part = jnp.zeros((1, H), jnp.float32)
        for k, aref in enumerate((adj_top_ref, adj_bot_ref)):
            ak = aref[0].astype(jnp.bfloat16)
            cache_scr[pl.ds(fi * BM + k * BS, BS), :] = ak
            tacc = jnp.dot(ak, s1_scr[...], preferred_element_type=jnp.float32)
            h1 = jnp.maximum(tacc + b1_ref[...], 0.0)
            s2_scr[fb, pl.ds(fi * BM + k * BS, BS), :] = jnp.dot(
                h1, w2_ref[...], preferred_element_type=jnp.float32
            ).astype(jnp.bfloat16)
            part = part + jnp.sum(h1, axis=0, keepdims=True)
        row = 2 * fb

        @pl.when(fi == 0)
        def _():
            gh_ref[pl.ds(row, 1), :] = part

        @pl.when(fi != 0)
        def _():
            gh_ref[pl.ds(row, 1), :] += part


@functools.partial(jax.jit, static_argnames=("interpret",))
def _encoder(x, adj, W1, b1, W2, b2, interpret=False):
    b1r = b1.reshape(1, H)
    b2r = b2.reshape(1, H)

    h2, gh = pl.pallas_call(
        _fused_kernel,
        grid=(3 * NI,),
        in_specs=[
            pl.BlockSpec(
                (1, BS, N),
                lambda t: (
                    jnp.where(t < NI, 0, 1),
                    jnp.where(
                        t < 2 * NI,
                        jnp.where(t < NI, t, t - NI) * NS,
                        (NI - 1) * NS,
                    ),
                    0,
                ),
            ),
            pl.BlockSpec(
                (1, BS, N),
                lambda t: (
                    jnp.where(t < NI, 0, 1),
                    jnp.where(
                        t < 2 * NI,
                        jnp.where(t < NI, t, t - NI) * NS,
                        (NI - 1) * NS,
                    )
                    + 1,
                    0,
                ),
            ),
            pl.BlockSpec((1, N, F), lambda t: (jnp.where(t < NI, 0, 1), 0, 0)),
            pl.BlockSpec((F, H), lambda t: (0, 0)),
            pl.BlockSpec((1, H), lambda t: (0, 0)),
            pl.BlockSpec((H, H), lambda t: (0, 0)),
            pl.BlockSpec((1, H), lambda t: (0, 0)),
        ],
        out_specs=[
            pl.BlockSpec(
                (1, BM, H),
                lambda t: (
                    jnp.where(t < 2 * NI, 0, 1),
                    jnp.where(
                        t < NI, 0, jnp.where(t < 2 * NI, t - NI, t - 2 * NI)
                    ),
                    0,
                ),
            ),
            pl.BlockSpec((4, H), lambda t: (0, 0)),
        ],
        out_shape=[
            jax.ShapeDtypeStruct((B, N, H), jnp.float32),
            jax.ShapeDtypeStruct((4, H), jnp.float32),
        ],
        scratch_shapes=[
            pltpu.VMEM((N, H), jnp.bfloat16),
            pltpu.VMEM((B, N, H), jnp.bfloat16),
            pltpu.VMEM((N, N), jnp.bfloat16),
        ],
        compiler_params=pltpu.CompilerParams(
            dimension_semantics=("arbitrary",),
            vmem_limit_bytes=100 * 1024 * 1024,
        ),
        interpret=interpret,
    )(adj, adj, x, W1, b1r, W2, b2r)

    return h2, gh.reshape(B, 2 * H)


def kernel(x, adj, W1, b1, W2, b2):
    return _encoder(x, adj, W1, b1, W2, b2)


# speedup vs baseline: 1.0217x; 1.0217x over previous
import functools

import jax
import jax.numpy as jnp
from jax.experimental import pallas as pl
from jax.experimental.pallas import tpu as pltpu

B, N, F, H = 2, 4096, 128, 128
BM = 512
NI = N // BM
BN = 1024
NJ = N // BN


def _fused_kernel(adj_ref, x_ref, w1_ref, b1_ref, w2_ref, b2_ref,
                  h2_ref, gh_ref, s1_scr, s2_scr, cache_scr):
    t = pl.program_id(0)

    @pl.when(((t >= NI) & (t < NI + NJ)) | (t >= 2 * NI))
    def _():
        pb = jnp.where(t < 2 * NI, 0, 1)
        pj = jnp.where(t < 2 * NI, t - NI, t - 2 * NI)
        a = cache_scr[pl.ds(pj * BN, BN), :]
        tacc = jnp.dot(a, s2_scr[pb], preferred_element_type=jnp.float32)
        h2 = jnp.maximum(tacc + b2_ref[...], 0.0)
        h2_ref[...] = h2[None]
        part = jnp.sum(h2, axis=0, keepdims=True)
        row = 2 * pb + 1

        @pl.when(pj == 0)
        def _():
            gh_ref[pl.ds(row, 1), :] = part

        @pl.when(pj != 0)
        def _():
            gh_ref[pl.ds(row, 1), :] += part

    @pl.when(t < 2 * NI)
    def _():
        fb = jnp.where(t < NI, 0, 1)
        fi = jnp.where(t < NI, t, t - NI)

        @pl.when(fi == 0)
        def _():
            s1 = jnp.dot(
                x_ref[0], w1_ref[...], preferred_element_type=jnp.float32
            )
            s1_scr[...] = s1.astype(jnp.bfloat16)

        a = adj_ref[0].astype(jnp.bfloat16)
        cache_scr[pl.ds(fi * BM, BM), :] = a
        tacc = jnp.dot(a, s1_scr[...], preferred_element_type=jnp.float32)
        h1 = jnp.maximum(tacc + b1_ref[...], 0.0)
        s2_scr[fb, pl.ds(fi * BM, BM), :] = jnp.dot(
            h1, w2_ref[...], preferred_element_type=jnp.float32
        ).astype(jnp.bfloat16)
        part = jnp.sum(h1, axis=0, keepdims=True)
        row = 2 * fb

        @pl.when(fi == 0)
        def _():
            gh_ref[pl.ds(row, 1), :] = part

        @pl.when(fi != 0)
        def _():
            gh_ref[pl.ds(row, 1), :] += part


@functools.partial(jax.jit, static_argnames=("interpret",))
def _encoder(x, adj, W1, b1, W2, b2, interpret=False):
    b1r = b1.reshape(1, H)
    b2r = b2.reshape(1, H)

    h2, gh = pl.pallas_call(
        _fused_kernel,
        grid=(2 * NI + NJ,),
        in_specs=[
            pl.BlockSpec(
                (1, BM, N),
                lambda t: (
                    jnp.where(t < NI, 0, 1),
                    jnp.where(t < 2 * NI, jnp.where(t < NI, t, t - NI), NI - 1),
                    0,
                ),
            ),
            pl.BlockSpec((1, N, F), lambda t: (jnp.where(t < NI, 0, 1), 0, 0)),
            pl.BlockSpec((F, H), lambda t: (0, 0)),
            pl.BlockSpec((1, H), lambda t: (0, 0)),
            pl.BlockSpec((H, H), lambda t: (0, 0)),
            pl.BlockSpec((1, H), lambda t: (0, 0)),
        ],
        out_specs=[
            pl.BlockSpec(
                (1, BN, H),
                lambda t: (
                    jnp.where(t < 2 * NI, 0, 1),
                    jnp.where(
                        t < NI,
                        0,
                        jnp.where(
                            t < NI + NJ,
                            t - NI,
                            jnp.where(t < 2 * NI, NJ - 1, t - 2 * NI),
                        ),
                    ),
                    0,
                ),
            ),
            pl.BlockSpec((4, H), lambda t: (0, 0)),
        ],
        out_shape=[
            jax.ShapeDtypeStruct((B, N, H), jnp.float32),
            jax.ShapeDtypeStruct((4, H), jnp.float32),
        ],
        scratch_shapes=[
            pltpu.VMEM((N, H), jnp.bfloat16),
            pltpu.VMEM((B, N, H), jnp.bfloat16),
            pltpu.VMEM((N, N), jnp.bfloat16),
        ],
        compiler_params=pltpu.CompilerParams(
            dimension_semantics=("arbitrary",),
            vmem_limit_bytes=100 * 1024 * 1024,
        ),
        interpret=interpret,
    )(adj, x, W1, b1r, W2, b2r)

    return h2, gh.reshape(B, 2 * H)


def kernel(x, adj, W1, b1, W2, b2):
    return _encoder(x, adj, W1, b1, W2, b2)


# speedup vs baseline: 1.0263x; 1.0045x over previous
import functools

import jax
import jax.numpy as jnp
from jax.experimental import pallas as pl
from jax.experimental.pallas import tpu as pltpu

B, N, F, H = 2, 4096, 128, 128
BM = 512
NI = N // BM
BN = 1024
NJ = N // BN


def _fused_kernel(adj_ref, x_ref, w1_ref, b1_ref, w2_ref, b2_ref,
                  h2_ref, gh_ref, s1_scr, s2_scr, cache_scr):
    t = pl.program_id(0)

    @pl.when(((t >= NI) & (t < NI + NJ)) | (t >= 2 * NI))
    def _():
        pb = jnp.where(t < 2 * NI, 0, 1)
        pj = jnp.where(t < 2 * NI, t - NI, t - 2 * NI)
        a = cache_scr[pl.ds(pj * BN, BN), :]
        tacc = jnp.dot(a, s2_scr[pb], preferred_element_type=jnp.float32)
        h2 = jnp.maximum(tacc + b2_ref[...], 0.0)
        h2_ref[...] = h2[None]
        part = jnp.sum(h2, axis=0, keepdims=True)
        row = 2 * pb + 1

        @pl.when(pj == 0)
        def _():
            gh_ref[pl.ds(row, 1), :] = part

        @pl.when(pj != 0)
        def _():
            gh_ref[pl.ds(row, 1), :] += part

    @pl.when(t < 2 * NI)
    def _():
        fb = jnp.where(t < NI, 0, 1)
        fi = jnp.where(t < NI, t, t - NI)

        @pl.when(fi == 0)
        def _():
            s1 = jnp.dot(
                x_ref[0], w1_ref[...], preferred_element_type=jnp.float32
            )
            s1_scr[...] = s1.astype(jnp.bfloat16)

        cache_scr[pl.ds(fi * BM, BM), :] = adj_ref[0].astype(jnp.bfloat16)
        a = cache_scr[pl.ds(fi * BM, BM), :]
        tacc = jnp.dot(a, s1_scr[...], preferred_element_type=jnp.float32)
        h1 = jnp.maximum(tacc + b1_ref[...], 0.0)
        s2_scr[fb, pl.ds(fi * BM, BM), :] = jnp.dot(
            h1, w2_ref[...], preferred_element_type=jnp.float32
        ).astype(jnp.bfloat16)
        part = jnp.sum(h1, axis=0, keepdims=True)
        row = 2 * fb

        @pl.when(fi == 0)
        def _():
            gh_ref[pl.ds(row, 1), :] = part

        @pl.when(fi != 0)
        def _():
            gh_ref[pl.ds(row, 1), :] += part


@functools.partial(jax.jit, static_argnames=("interpret",))
def _encoder(x, adj, W1, b1, W2, b2, interpret=False):
    b1r = b1.reshape(1, H)
    b2r = b2.reshape(1, H)

    h2, gh = pl.pallas_call(
        _fused_kernel,
        grid=(2 * NI + NJ,),
        in_specs=[
            pl.BlockSpec(
                (1, BM, N),
                lambda t: (
                    jnp.where(t < NI, 0, 1),
                    jnp.where(t < 2 * NI, jnp.where(t < NI, t, t - NI), NI - 1),
                    0,
                ),
            ),
            pl.BlockSpec((1, N, F), lambda t: (jnp.where(t < NI, 0, 1), 0, 0)),
            pl.BlockSpec((F, H), lambda t: (0, 0)),
            pl.BlockSpec((1, H), lambda t: (0, 0)),
            pl.BlockSpec((H, H), lambda t: (0, 0)),
            pl.BlockSpec((1, H), lambda t: (0, 0)),
        ],
        out_specs=[
            pl.BlockSpec(
                (1, BN, H),
                lambda t: (
                    jnp.where(t < 2 * NI, 0, 1),
                    jnp.where(
                        t < NI,
                        0,
                        jnp.where(
                            t < NI + NJ,
                            t - NI,
                            jnp.where(t < 2 * NI, NJ - 1, t - 2 * NI),
                        ),
                    ),
                    0,
                ),
            ),
            pl.BlockSpec((4, H), lambda t: (0, 0)),
        ],
        out_shape=[
            jax.ShapeDtypeStruct((B, N, H), jnp.float32),
            jax.ShapeDtypeStruct((4, H), jnp.float32),
        ],
        scratch_shapes=[
            pltpu.VMEM((N, H), jnp.bfloat16),
            pltpu.VMEM((B, N, H), jnp.bfloat16),
            pltpu.VMEM((N, N), jnp.bfloat16),
        ],
        compiler_params=pltpu.CompilerParams(
            dimension_semantics=("arbitrary",),
            vmem_limit_bytes=100 * 1024 * 1024,
        ),
        interpret=interpret,
    )(adj, x, W1, b1r, W2, b2r)

    return h2, gh.reshape(B, 2 * H)


def kernel(x, adj, W1, b1, W2, b2):
    return _encoder(x, adj, W1, b1, W2, b2)
